# baseline (speedup 1.0000x reference)
import functools

import jax
import jax.numpy as jnp
from jax import lax
from jax.experimental import pallas as pl
from jax.experimental.pallas import tpu as pltpu
from jax.experimental.pallas import tpu_sc as plsc

NUM_SYMBOLS = 1000000
D = 64
B_QUERY = 16384
B_SUPPORT = 128
NW = 32
L = 16
PAIRS_PER_W = B_QUERY // NW
CHUNK_PAIRS = 16
CHUNK_ROWS = 2 * CHUNK_PAIRS
NQCHUNK = PAIRS_PER_W // CHUNK_PAIRS
NSCHUNK = 2 * B_SUPPORT // CHUNK_ROWS
TROWS = 8
QIDX = PAIRS_PER_W * 2
EPS = 1e-8
EPS2 = EPS * EPS


def _rsqrt16(x):
  xb = plsc.bitcast(x, jnp.int32)
  yb = jnp.full((L,), 0x5F3759DF, jnp.int32) - lax.shift_right_logical(xb, 1)
  y = plsc.bitcast(yb, jnp.float32)
  for _ in range(3):
    y = y * (1.5 - 0.5 * x * y * y)
  return y


def _inv_clamped_norm(ssq, inv_eps):
  guarded = jnp.maximum(ssq, jnp.float32(EPS2))
  return jnp.where(ssq <= EPS2, inv_eps, _rsqrt16(guarded))


def _issue_chunk(table_hbm, idx_v, buf, sem, pos):
  for g in range(CHUNK_ROWS // L):
    tv = lax.shift_right_logical(idx_v[pl.ds(pos + g * L, L)], 3)
    for j in range(L):
      t = tv[j]
      pltpu.async_copy(
          table_hbm.at[pl.ds(t * TROWS, TROWS)],
          buf.at[pl.ds((g * L + j) * TROWS, TROWS)], sem)


def _wait_chunk(table_hbm, buf, sem):
  pltpu.make_async_copy(
      table_hbm.at[pl.ds(0, CHUNK_ROWS * TROWS)], buf, sem).wait()


def _splat_row(idx_v, flat_pos, slab_row):
  iv = plsc.load_gather(idx_v, [jnp.full((L,), flat_pos, jnp.int32)])
  return jnp.full((L,), slab_row * TROWS, jnp.int32) + lax.bitwise_and(iv, 7)


def _sc_body(table_hbm, qidx_hbm, sidx_hbm, out_hbm,
             idx_v, buf0, buf1, num_v, sq_v, pacc_n, pacc_s,
             sem0, sem1):
  wid = lax.axis_index("s") * 2 + lax.axis_index("c")
  iota = lax.iota(jnp.int32, L)
  iota16 = [iota + L * k for k in range(4)]
  zero = jnp.zeros((L,), jnp.float32)
  bufs = (buf0, buf1)
  sems = (sem0, sem1)

  pltpu.sync_copy(qidx_hbm.at[pl.ds(wid * QIDX, QIDX)], idx_v.at[pl.ds(0, QIDX)])
  pltpu.sync_copy(sidx_hbm, idx_v.at[pl.ds(QIDX, 2 * B_SUPPORT)])

  _issue_chunk(table_hbm, idx_v, buf0, sem0, QIDX)
  _issue_chunk(table_hbm, idx_v, buf1, sem1, QIDX + CHUNK_ROWS)

  acc = (zero,) * 8
  for blk in range(NSCHUNK):
    b = blk % 2
    _wait_chunk(table_hbm, bufs[b], sems[b])
    buf = bufs[b]
    pos = QIDX + blk * CHUNK_ROWS

    def sbody(p, a, buf=buf, pos=pos):
      re = _splat_row(idx_v, pos + 2 * p, 2 * p)
      ro = _splat_row(idx_v, pos + 2 * p + 1, 2 * p + 1)
      out = []
      for k in range(4):
        out.append(a[k] + plsc.load_gather(buf, [re, iota16[k]]))
      for k in range(4):
        out.append(a[4 + k] + plsc.load_gather(buf, [ro, iota16[k]]))
      return tuple(out)

    acc = lax.fori_loop(0, CHUNK_PAIRS, sbody, acc)
    nxt = blk + 2
    if nxt < NSCHUNK:
      _issue_chunk(table_hbm, idx_v, bufs[b], sems[b], QIDX + nxt * CHUNK_ROWS)
    else:
      _issue_chunk(table_hbm, idx_v, bufs[b], sems[b],
                   (nxt - NSCHUNK) * CHUNK_ROWS)

  inv = jnp.float32(1.0 / B_SUPPORT)
  smA = [acc[k] * inv for k in range(4)]
  smB = [acc[4 + k] * inv for k in range(4)]

  def qloop(i, carry):
    for b in range(2):
      c = 2 * i + b
      _wait_chunk(table_hbm, bufs[b], sems[b])
      buf = bufs[b]
      pos = c * CHUNK_ROWS

      def pbody(p, inner):
        re = _splat_row(idx_v, pos + 2 * p, 2 * p)
        ro = _splat_row(idx_v, pos + 2 * p + 1, 2 * p + 1)
        n = zero
        s = zero
        for k in range(4):
          ve = plsc.load_gather(buf, [re, iota16[k]])
          vo = plsc.load_gather(buf, [ro, iota16[k]])
          n = n + ve * smA[k] + vo * smB[k]
          s = s + ve * ve + vo * vo
        pacc_n[pl.ds(p * L, L)] = n
        pacc_s[pl.ds(p * L, L)] = s
        return inner

      lax.fori_loop(0, CHUNK_PAIRS, pbody, 0, unroll=4)

      @pl.when(c + 2 < NQCHUNK)
      def _():
        _issue_chunk(table_hbm, idx_v, bufs[b], sems[b],
                     (c + 2) * CHUNK_ROWS)

      nsum = zero
      ssum = zero
      for j in range(L):
        nsum = nsum + plsc.load_gather(pacc_n, [iota * L + j])
        ssum = ssum + plsc.load_gather(pacc_s, [iota * L + j])
      num_v[pl.ds(c * L, L)] = nsum
      sq_v[pl.ds(c * L, L)] = ssum
    return carry

  lax.fori_loop(0, NQCHUNK // 2, qloop, 0)

  inv_eps = jnp.full((L,), 1.0 / EPS, jnp.float32)
  smsq16 = zero
  for k in range(4):
    smsq16 = smsq16 + smA[k] * smA[k] + smB[k] * smB[k]
  smsq = jnp.full((L,), jnp.sum(smsq16), jnp.float32)
  inv2 = _inv_clamped_norm(smsq, inv_eps)

  def nbody(g, carry):
    n16 = num_v[pl.ds(g * L, L)]
    s16 = sq_v[pl.ds(g * L, L)]
    inv1 = _inv_clamped_norm(s16, inv_eps)
    num_v[pl.ds(g * L, L)] = n16 * inv1 * inv2
    return carry

  lax.fori_loop(0, PAIRS_PER_W // L, nbody, 0)
  pltpu.sync_copy(num_v, out_hbm.at[pl.ds(wid * PAIRS_PER_W, PAIRS_PER_W)])


_sc_matcher = functools.partial(
    pl.kernel,
    mesh=plsc.VectorSubcoreMesh(core_axis_name="c", subcore_axis_name="s"),
    out_type=jax.ShapeDtypeStruct((B_QUERY,), jnp.float32),
    scratch_types=[
        pltpu.VMEM((QIDX + 2 * B_SUPPORT,), jnp.int32),
        pltpu.VMEM((CHUNK_ROWS * TROWS, D), jnp.float32),
        pltpu.VMEM((CHUNK_ROWS * TROWS, D), jnp.float32),
        pltpu.VMEM((PAIRS_PER_W,), jnp.float32),
        pltpu.VMEM((PAIRS_PER_W,), jnp.float32),
        pltpu.VMEM((CHUNK_PAIRS * L,), jnp.float32),
        pltpu.VMEM((CHUNK_PAIRS * L,), jnp.float32),
        pltpu.SemaphoreType.DMA,
        pltpu.SemaphoreType.DMA,
    ],
    compiler_params=pltpu.CompilerParams(needs_layout_passes=False),
)(_sc_body)


_TBW = 36864


def _transpose_body(in_ref, out_ref):
  out_ref[...] = in_ref[...].T


def _relayout_table(tableT):
  n = tableT.shape[1]
  grid = (n + _TBW - 1) // _TBW
  return pl.pallas_call(
      _transpose_body,
      grid=(grid,),
      in_specs=[pl.BlockSpec((D, _TBW), lambda i: (0, i))],
      out_specs=pl.BlockSpec((_TBW, D), lambda i: (i, 0)),
      out_shape=jax.ShapeDtypeStruct((n, D), jnp.float32),
  )(tableT)


def kernel(query, support, symbol_emb):
  qflat = query.reshape(-1).astype(jnp.int32)
  sflat = support.reshape(-1).astype(jnp.int32)
  table_rm = _relayout_table(symbol_emb.T)
  return _sc_matcher(table_rm, qflat, sflat)


# speedup vs baseline: 1.0014x; 1.0014x over previous
import functools

import jax
import jax.numpy as jnp
from jax import lax
from jax.experimental import pallas as pl
from jax.experimental.pallas import tpu as pltpu
from jax.experimental.pallas import tpu_sc as plsc

NUM_SYMBOLS = 1000000
D = 64
B_QUERY = 16384
B_SUPPORT = 128
NW = 32
L = 16
PAIRS_PER_W = B_QUERY // NW
CHUNK_PAIRS = 16
CHUNK_ROWS = 2 * CHUNK_PAIRS
NQCHUNK = PAIRS_PER_W // CHUNK_PAIRS
NSCHUNK = 2 * B_SUPPORT // CHUNK_ROWS
TROWS = 8
QIDX = PAIRS_PER_W * 2
EPS = 1e-8
EPS2 = EPS * EPS


def _rsqrt16(x):
  xb = plsc.bitcast(x, jnp.int32)
  yb = jnp.full((L,), 0x5F3759DF, jnp.int32) - lax.shift_right_logical(xb, 1)
  y = plsc.bitcast(yb, jnp.float32)
  for _ in range(3):
    y = y * (1.5 - 0.5 * x * y * y)
  return y


def _inv_clamped_norm(ssq, inv_eps):
  guarded = jnp.maximum(ssq, jnp.float32(EPS2))
  return jnp.where(ssq <= EPS2, inv_eps, _rsqrt16(guarded))


def _issue_chunk(table_hbm, idx_v, buf, sem, pos):
  for g in range(CHUNK_ROWS // L):
    tv = lax.shift_right_logical(idx_v[pl.ds(pos + g * L, L)], 3)
    for j in range(L):
      t = tv[j]
      pltpu.async_copy(
          table_hbm.at[pl.ds(t * TROWS, TROWS)],
          buf.at[pl.ds((g * L + j) * TROWS, TROWS)], sem)


def _wait_chunk(table_hbm, buf, sem):
  pltpu.make_async_copy(
      table_hbm.at[pl.ds(0, CHUNK_ROWS * TROWS)], buf, sem).wait()


def _splat_row(idx_v, flat_pos, slab_row):
  iv = plsc.load_gather(idx_v, [jnp.full((L,), flat_pos, jnp.int32)])
  return jnp.full((L,), slab_row * TROWS, jnp.int32) + lax.bitwise_and(iv, 7)


def _sc_body(table_hbm, qidx_hbm, sidx_hbm, out_hbm,
             idx_v, buf0, buf1, num_v, sq_v, pacc_n, pacc_s,
             sem0, sem1):
  wid = lax.axis_index("s") * 2 + lax.axis_index("c")
  iota = lax.iota(jnp.int32, L)
  iota16 = [iota + L * k for k in range(4)]
  zero = jnp.zeros((L,), jnp.float32)
  bufs = (buf0, buf1)
  sems = (sem0, sem1)

  pltpu.sync_copy(qidx_hbm.at[pl.ds(wid * QIDX, QIDX)], idx_v.at[pl.ds(0, QIDX)])
  pltpu.sync_copy(sidx_hbm, idx_v.at[pl.ds(QIDX, 2 * B_SUPPORT)])

  _issue_chunk(table_hbm, idx_v, buf0, sem0, QIDX)
  _issue_chunk(table_hbm, idx_v, buf1, sem1, QIDX + CHUNK_ROWS)

  acc = (zero,) * 8
  for blk in range(NSCHUNK):
    b = blk % 2
    _wait_chunk(table_hbm, bufs[b], sems[b])
    buf = bufs[b]
    pos = QIDX + blk * CHUNK_ROWS

    def sbody(p, a, buf=buf, pos=pos):
      re = _splat_row(idx_v, pos + 2 * p, 2 * p)
      ro = _splat_row(idx_v, pos + 2 * p + 1, 2 * p + 1)
      out = []
      for k in range(4):
        out.append(a[k] + plsc.load_gather(buf, [re, iota16[k]]))
      for k in range(4):
        out.append(a[4 + k] + plsc.load_gather(buf, [ro, iota16[k]]))
      return tuple(out)

    acc = lax.fori_loop(0, CHUNK_PAIRS, sbody, acc)
    nxt = blk + 2
    if nxt < NSCHUNK:
      _issue_chunk(table_hbm, idx_v, bufs[b], sems[b], QIDX + nxt * CHUNK_ROWS)
    else:
      _issue_chunk(table_hbm, idx_v, bufs[b], sems[b],
                   (nxt - NSCHUNK) * CHUNK_ROWS)

  inv = jnp.float32(1.0 / B_SUPPORT)
  smA = [acc[k] * inv for k in range(4)]
  smB = [acc[4 + k] * inv for k in range(4)]

  def qloop(i, carry):
    for b in range(2):
      c = 2 * i + b
      _wait_chunk(table_hbm, bufs[b], sems[b])
      buf = bufs[b]
      pos = c * CHUNK_ROWS

      def pbody(p, inner):
        re = _splat_row(idx_v, pos + 2 * p, 2 * p)
        ro = _splat_row(idx_v, pos + 2 * p + 1, 2 * p + 1)
        n = zero
        s = zero
        for k in range(4):
          ve = plsc.load_gather(buf, [re, iota16[k]])
          vo = plsc.load_gather(buf, [ro, iota16[k]])
          n = n + ve * smA[k] + vo * smB[k]
          s = s + ve * ve + vo * vo
        pacc_n[pl.ds(p * L, L)] = n
        pacc_s[pl.ds(p * L, L)] = s
        return inner

      lax.fori_loop(0, CHUNK_PAIRS, pbody, 0)

      @pl.when(c + 2 < NQCHUNK)
      def _():
        _issue_chunk(table_hbm, idx_v, bufs[b], sems[b],
                     (c + 2) * CHUNK_ROWS)

      nsum = zero
      ssum = zero
      for j in range(L):
        nsum = nsum + plsc.load_gather(pacc_n, [iota * L + j])
        ssum = ssum + plsc.load_gather(pacc_s, [iota * L + j])
      num_v[pl.ds(c * L, L)] = nsum
      sq_v[pl.ds(c * L, L)] = ssum
    return carry

  lax.fori_loop(0, NQCHUNK // 2, qloop, 0)

  inv_eps = jnp.full((L,), 1.0 / EPS, jnp.float32)
  smsq16 = zero
  for k in range(4):
    smsq16 = smsq16 + smA[k] * smA[k] + smB[k] * smB[k]
  smsq = jnp.full((L,), jnp.sum(smsq16), jnp.float32)
  inv2 = _inv_clamped_norm(smsq, inv_eps)

  def nbody(g, carry):
    n16 = num_v[pl.ds(g * L, L)]
    s16 = sq_v[pl.ds(g * L, L)]
    inv1 = _inv_clamped_norm(s16, inv_eps)
    num_v[pl.ds(g * L, L)] = n16 * inv1 * inv2
    return carry

  lax.fori_loop(0, PAIRS_PER_W // L, nbody, 0)
  pltpu.sync_copy(num_v, out_hbm.at[pl.ds(wid * PAIRS_PER_W, PAIRS_PER_W)])


_sc_matcher = functools.partial(
    pl.kernel,
    mesh=plsc.VectorSubcoreMesh(core_axis_name="c", subcore_axis_name="s"),
    out_type=jax.ShapeDtypeStruct((B_QUERY,), jnp.float32),
    scratch_types=[
        pltpu.VMEM((QIDX + 2 * B_SUPPORT,), jnp.int32),
        pltpu.VMEM((CHUNK_ROWS * TROWS, D), jnp.float32),
        pltpu.VMEM((CHUNK_ROWS * TROWS, D), jnp.float32),
        pltpu.VMEM((PAIRS_PER_W,), jnp.float32),
        pltpu.VMEM((PAIRS_PER_W,), jnp.float32),
        pltpu.VMEM((CHUNK_PAIRS * L,), jnp.float32),
        pltpu.VMEM((CHUNK_PAIRS * L,), jnp.float32),
        pltpu.SemaphoreType.DMA,
        pltpu.SemaphoreType.DMA,
    ],
    compiler_params=pltpu.CompilerParams(needs_layout_passes=False),
)(_sc_body)


_TBW = 36864


def _transpose_body(in_ref, out_ref):
  out_ref[...] = in_ref[...].T


def _relayout_table(tableT):
  n = tableT.shape[1]
  grid = (n + _TBW - 1) // _TBW
  return pl.pallas_call(
      _transpose_body,
      grid=(grid,),
      in_specs=[pl.BlockSpec((D, _TBW), lambda i: (0, i))],
      out_specs=pl.BlockSpec((_TBW, D), lambda i: (i, 0)),
      out_shape=jax.ShapeDtypeStruct((n, D), jnp.float32),
  )(tableT)


def kernel(query, support, symbol_emb):
  qflat = query.reshape(-1).astype(jnp.int32)
  sflat = support.reshape(-1).astype(jnp.int32)
  table_rm = _relayout_table(symbol_emb.T)
  return _sc_matcher(table_rm, qflat, sflat)
